# Initial kernel scaffold; baseline (speedup 1.0000x reference)
import jax
import jax.numpy as jnp
from jax.experimental import pallas as pl


def kernel(samples, map_node_values, n):
    raise NotImplementedError("write your pallas kernel here")


# speedup vs baseline: 2.2040x; 2.2040x over previous
import jax
import jax.numpy as jnp
from jax import lax
from jax.experimental import pallas as pl
from jax.experimental.pallas import tpu as pltpu
from jax.experimental.pallas import tpu_sc as plsc

K = 1000000
D = 16
NTILES = 32
CHUNK_ROWS = 1200
BLOCKS = CHUNK_ROWS // 16
NFULL = K // CHUNK_ROWS
TAIL_START = K - CHUNK_ROWS
TAIL_BLOCK0 = (NFULL * CHUNK_ROWS - TAIL_START) // 16
TAIL_TILE = NFULL % NTILES


def _iota16():
    return lax.iota(jnp.int32, 16)


def _lane_gather(v, idx16):
    dn = lax.GatherDimensionNumbers(
        offset_dims=(), collapsed_slice_dims=(0,), start_index_map=(0,))
    return lax.gather(v, idx16[:, None], dn, slice_sizes=(1,),
                      mode=lax.GatherScatterMode.PROMISE_IN_BOUNDS)


def _splat(v, lane):
    return _lane_gather(v, jnp.full((16,), lane, jnp.int32))


def _merge_topk(cur_v, cur_i, blk_v, blk_i):
    sk, si = plsc.sort_key_val(blk_v, blk_i)
    rv = lax.rev(sk, (0,))
    ri = lax.rev(si, (0,))
    take_a = cur_v <= rv
    mv = jnp.where(take_a, cur_v, rv)
    mi = jnp.where(take_a, cur_i, ri)
    nv, ni = plsc.sort_key_val(mv, mi)
    return nv, ni, _splat(nv, 7)


def _sc_body(mnv_hbm, smp_hbm, outv_hbm, outi_hbm, buf, sbuf, resv, resi):
    cid = lax.axis_index("c")
    sid = lax.axis_index("s")
    wid = sid * 2 + cid

    pltpu.sync_copy(smp_hbm, sbuf)
    s = sbuf[...]
    s_d = [_splat(s, d) for d in range(D)]

    iota = _iota16()
    inf = jnp.float32(jnp.inf)
    base_ids = [iota * 16 + d for d in range(D)]

    def block_dist2(row_ids0):
        acc = jnp.zeros((16,), jnp.float32)
        for d in range(D):
            col = plsc.load_gather(buf, [row_ids0 + base_ids[d]])
            diff = col - s_d[d]
            acc = acc + diff * diff
        return acc

    def consider(acc, gidx, cur_v, cur_i, tv):
        hit = jnp.any(acc < tv)

        def do(acc, gidx, cur_v, cur_i, tv):
            return _merge_topk(cur_v, cur_i, acc, gidx)

        def skip(acc, gidx, cur_v, cur_i, tv):
            return cur_v, cur_i, tv

        return lax.cond(hit, do, skip, acc, gidx, cur_v, cur_i, tv)

    def make_blk_body(start):
        def blk_body(b, carry):
            cur_v, cur_i, tv = carry
            acc = block_dist2(b * 256)
            gidx = start + b * 16 + iota
            return consider(acc, gidx, cur_v, cur_i, tv)
        return blk_body

    def chunk_body(j, carry):
        ch = wid + j * NTILES
        start = pl.multiple_of(ch * CHUNK_ROWS, 8)
        pltpu.sync_copy(mnv_hbm.at[pl.ds(start * D, CHUNK_ROWS * D)], buf)
        return lax.fori_loop(0, BLOCKS, make_blk_body(start), carry)

    init = (jnp.full((16,), inf), jnp.zeros((16,), jnp.int32),
            jnp.full((16,), inf))
    nchunks = (NFULL - wid + NTILES - 1) // NTILES
    cur_v, cur_i, tv = lax.fori_loop(0, nchunks, chunk_body, init)

    @pl.when(wid == TAIL_TILE)
    def _():
        pltpu.sync_copy(
            mnv_hbm.at[pl.ds(TAIL_START * D, CHUNK_ROWS * D)], buf)
        cv, ci, _t = lax.fori_loop(
            TAIL_BLOCK0, BLOCKS, make_blk_body(TAIL_START), (cur_v, cur_i, tv))
        resv[...] = cv
        resi[...] = ci

    @pl.when(wid != TAIL_TILE)
    def _():
        resv[...] = cur_v
        resi[...] = cur_i

    pltpu.sync_copy(resv, outv_hbm.at[pl.ds(wid * 16, 16)])
    pltpu.sync_copy(resi, outi_hbm.at[pl.ds(wid * 16, 16)])


def _sc_dist_topk(mnv, smp):
    mesh = plsc.VectorSubcoreMesh(core_axis_name="c", subcore_axis_name="s")
    f = pl.kernel(
        _sc_body,
        mesh=mesh,
        compiler_params=pltpu.CompilerParams(needs_layout_passes=False),
        out_type=[
            jax.ShapeDtypeStruct((NTILES * 16,), jnp.float32),
            jax.ShapeDtypeStruct((NTILES * 16,), jnp.int32),
        ],
        scratch_types=[
            pltpu.VMEM((CHUNK_ROWS * D,), jnp.float32),
            pltpu.VMEM((16,), jnp.float32),
            pltpu.VMEM((16,), jnp.float32),
            pltpu.VMEM((16,), jnp.int32),
        ],
    )
    return f(mnv, smp)


def _tc_merge_body(v_ref, i_ref, idx_ref, val_ref):
    V = v_ref[...]
    I = i_ref[...]
    big = jnp.int32(2**31 - 1)
    inf = jnp.float32(jnp.inf)
    idxs = []
    vals = []
    for _ in range(8):
        m = jnp.min(V)
        sel = V == m
        ci = jnp.min(jnp.where(sel, I, big))
        idxs.append(ci)
        vals.append(m)
        V = jnp.where(sel & (I == ci), inf, V)
    idx_ref[...] = jnp.stack(idxs)
    val_ref[...] = jnp.sqrt(jnp.stack(vals))


def _tc_merge(v2d, i2d):
    return pl.pallas_call(
        _tc_merge_body,
        out_shape=[
            jax.ShapeDtypeStruct((8,), jnp.int32),
            jax.ShapeDtypeStruct((8,), jnp.float32),
        ],
    )(v2d, i2d)


def kernel(samples, map_node_values, n):
    cv, ci = _sc_dist_topk(map_node_values.reshape(-1), samples)
    idx, vals = _tc_merge(cv.reshape(NTILES, 16), ci.reshape(NTILES, 16))
    return idx, vals


# speedup vs baseline: 2.3637x; 1.0725x over previous
import jax
import jax.numpy as jnp
from jax import lax
from jax.experimental import pallas as pl
from jax.experimental.pallas import tpu as pltpu
from jax.experimental.pallas import tpu_sc as plsc

K = 1000000
D = 16
NTILES = 32
CHUNK_ROWS = 1200
BLOCKS = CHUNK_ROWS // 16
NFULL = K // CHUNK_ROWS
TAIL_START = K - CHUNK_ROWS
TAIL_BLOCK0 = (NFULL * CHUNK_ROWS - TAIL_START) // 16
TAIL_TILE = NFULL % NTILES


def _iota16():
    return lax.iota(jnp.int32, 16)


def _lane_gather(v, idx16):
    dn = lax.GatherDimensionNumbers(
        offset_dims=(), collapsed_slice_dims=(0,), start_index_map=(0,))
    return lax.gather(v, idx16[:, None], dn, slice_sizes=(1,),
                      mode=lax.GatherScatterMode.PROMISE_IN_BOUNDS)


def _splat(v, lane):
    return _lane_gather(v, jnp.full((16,), lane, jnp.int32))


def _merge_topk(cur_v, cur_i, blk_v, blk_i):
    sk, si = plsc.sort_key_val(blk_v, blk_i)
    rv = lax.rev(sk, (0,))
    ri = lax.rev(si, (0,))
    take_a = cur_v <= rv
    mv = jnp.where(take_a, cur_v, rv)
    mi = jnp.where(take_a, cur_i, ri)
    nv, ni = plsc.sort_key_val(mv, mi)
    return nv, ni, _splat(nv, 7)


def _sc_body(mnv_hbm, smp_hbm, outv_hbm, outi_hbm, buf, sbuf, resv, resi):
    cid = lax.axis_index("c")
    sid = lax.axis_index("s")
    wid = sid * 2 + cid

    pltpu.sync_copy(smp_hbm, sbuf)
    s = sbuf[...]
    iota = _iota16()
    inf = jnp.float32(jnp.inf)
    diag = [(iota + d) & 15 for d in range(D)]
    s_d = [_lane_gather(s, diag[d]) for d in range(D)]
    base_ids = [iota * 16 + diag[d] for d in range(D)]

    def block_dist2(row_ids0):
        acc = jnp.zeros((16,), jnp.float32)
        for d in range(D):
            col = plsc.load_gather(buf, [row_ids0 + base_ids[d]])
            diff = col - s_d[d]
            acc = acc + diff * diff
        return acc

    def consider(acc, gidx, cur_v, cur_i, tv):
        hit = jnp.any(acc < tv)

        def do(acc, gidx, cur_v, cur_i, tv):
            return _merge_topk(cur_v, cur_i, acc, gidx)

        def skip(acc, gidx, cur_v, cur_i, tv):
            return cur_v, cur_i, tv

        return lax.cond(hit, do, skip, acc, gidx, cur_v, cur_i, tv)

    def make_blk_body(start):
        def blk_body(b, carry):
            cur_v, cur_i, tv = carry
            acc = block_dist2(b * 256)
            gidx = start + b * 16 + iota
            return consider(acc, gidx, cur_v, cur_i, tv)
        return blk_body

    def chunk_body(j, carry):
        ch = wid + j * NTILES
        start = pl.multiple_of(ch * CHUNK_ROWS, 8)
        pltpu.sync_copy(mnv_hbm.at[pl.ds(start * D, CHUNK_ROWS * D)], buf)
        return lax.fori_loop(0, BLOCKS, make_blk_body(start), carry)

    init = (jnp.full((16,), inf), jnp.zeros((16,), jnp.int32),
            jnp.full((16,), inf))
    nchunks = (NFULL - wid + NTILES - 1) // NTILES
    cur_v, cur_i, tv = lax.fori_loop(0, nchunks, chunk_body, init)

    @pl.when(wid == TAIL_TILE)
    def _():
        pltpu.sync_copy(
            mnv_hbm.at[pl.ds(TAIL_START * D, CHUNK_ROWS * D)], buf)
        cv, ci, _t = lax.fori_loop(
            TAIL_BLOCK0, BLOCKS, make_blk_body(TAIL_START), (cur_v, cur_i, tv))
        resv[...] = cv
        resi[...] = ci

    @pl.when(wid != TAIL_TILE)
    def _():
        resv[...] = cur_v
        resi[...] = cur_i

    pltpu.sync_copy(resv, outv_hbm.at[pl.ds(wid * 16, 16)])
    pltpu.sync_copy(resi, outi_hbm.at[pl.ds(wid * 16, 16)])


def _sc_dist_topk(mnv, smp):
    mesh = plsc.VectorSubcoreMesh(core_axis_name="c", subcore_axis_name="s")
    f = pl.kernel(
        _sc_body,
        mesh=mesh,
        compiler_params=pltpu.CompilerParams(needs_layout_passes=False),
        out_type=[
            jax.ShapeDtypeStruct((NTILES * 16,), jnp.float32),
            jax.ShapeDtypeStruct((NTILES * 16,), jnp.int32),
        ],
        scratch_types=[
            pltpu.VMEM((CHUNK_ROWS * D,), jnp.float32),
            pltpu.VMEM((16,), jnp.float32),
            pltpu.VMEM((16,), jnp.float32),
            pltpu.VMEM((16,), jnp.int32),
        ],
    )
    return f(mnv, smp)


def _tc_merge_body(v_ref, i_ref, idx_ref, val_ref):
    V = v_ref[...]
    I = i_ref[...]
    big = jnp.int32(2**31 - 1)
    inf = jnp.float32(jnp.inf)
    idxs = []
    vals = []
    for _ in range(8):
        m = jnp.min(V)
        sel = V == m
        ci = jnp.min(jnp.where(sel, I, big))
        idxs.append(ci)
        vals.append(m)
        V = jnp.where(sel & (I == ci), inf, V)
    idx_ref[...] = jnp.stack(idxs)
    val_ref[...] = jnp.sqrt(jnp.stack(vals))


def _tc_merge(v2d, i2d):
    return pl.pallas_call(
        _tc_merge_body,
        out_shape=[
            jax.ShapeDtypeStruct((8,), jnp.int32),
            jax.ShapeDtypeStruct((8,), jnp.float32),
        ],
    )(v2d, i2d)


def kernel(samples, map_node_values, n):
    cv, ci = _sc_dist_topk(map_node_values.reshape(-1), samples)
    idx, vals = _tc_merge(cv.reshape(NTILES, 16), ci.reshape(NTILES, 16))
    return idx, vals
